# Initial kernel scaffold; baseline (speedup 1.0000x reference)
import jax
import jax.numpy as jnp
from jax.experimental import pallas as pl


def kernel(x):
    raise NotImplementedError("write your pallas kernel here")


# speedup vs baseline: 5.4753x; 5.4753x over previous
import jax
import jax.numpy as jnp
from jax.experimental import pallas as pl
from jax.experimental.pallas import tpu as pltpu

_CONF_THRESH = 0.25
_IOU_THRESH = 0.45
_MAX_PER_IMG = 100
_N = 5000
_NC = 80
_R = 40
_L = 128
_NP = _R * _L
_NEG = float("-inf")
_K = 4


def _setup_one(x_ref, k):
    cx = x_ref[k, 0]
    cy = x_ref[k, 1]
    w = x_ref[k, 2]
    h = x_ref[k, 3]
    obj = x_ref[k, 4]

    x1 = cx - w / 2.0
    y1 = cy - h / 2.0
    x2 = cx + w / 2.0
    y2 = cy + h / 2.0
    area = (x2 - x1) * (y2 - y1)

    cls = x_ref[k, 5:5 + _NC]
    ccf = jnp.max(cls, axis=0)
    cidx = jax.lax.broadcasted_iota(jnp.int32, (_NC, _R, _L), 0)
    cpred = jnp.min(jnp.where(cls == ccf[None], cidx, 2**30), axis=0).astype(jnp.float32)

    score = obj * ccf
    ri = jax.lax.broadcasted_iota(jnp.int32, (_R, _L), 0)
    li = jax.lax.broadcasted_iota(jnp.int32, (_R, _L), 1)
    flat = ri * _L + li
    valid = flat < _N

    m0 = jnp.max(jnp.where(valid, score, _NEG), keepdims=True)
    conf = jnp.minimum(_CONF_THRESH, m0)
    s0 = jnp.where(valid & (score >= conf), score, _NEG)
    return dict(x1=x1, y1=y1, x2=x2, y2=y2, area=area, obj=obj, ccf=ccf,
                cpred=cpred, flat=flat, s0=s0)


def _nms_body(x_ref, o_ref):
    imgs = [_setup_one(x_ref, k) for k in range(_K)]
    lane8 = jax.lax.broadcasted_iota(jnp.int32, (1, 8), 1)

    def step(t, ss):
        out = []
        for k in range(_K):
            d = imgs[k]
            s = ss[k]
            m = jnp.max(s, keepdims=True)
            i = jnp.min(jnp.where(s == m, d["flat"], 2**30), keepdims=True)
            pick = d["flat"] == i
            pf = pick.astype(jnp.float32)
            bx1 = jnp.sum(pf * d["x1"], keepdims=True)
            by1 = jnp.sum(pf * d["y1"], keepdims=True)
            bx2 = jnp.sum(pf * d["x2"], keepdims=True)
            by2 = jnp.sum(pf * d["y2"], keepdims=True)
            bobj = jnp.sum(pf * d["obj"], keepdims=True)
            bccf = jnp.sum(pf * d["ccf"], keepdims=True)
            bcls = jnp.sum(pf * d["cpred"], keepdims=True)

            okf = jnp.where(m > _NEG, 1.0, 0.0)
            row = (jnp.where(lane8 == 0, bx1, 0.0)
                   + jnp.where(lane8 == 1, by1, 0.0)
                   + jnp.where(lane8 == 2, bx2, 0.0)
                   + jnp.where(lane8 == 3, by2, 0.0)
                   + jnp.where(lane8 == 4, bobj, 0.0)
                   + jnp.where(lane8 == 5, bccf, 0.0)
                   + jnp.where(lane8 == 6, bcls, 0.0)) * okf
            o_ref[k, pl.ds(t, 1), :] = row

            xx1 = jnp.maximum(bx1, d["x1"])
            yy1 = jnp.maximum(by1, d["y1"])
            xx2 = jnp.minimum(bx2, d["x2"])
            yy2 = jnp.minimum(by2, d["y2"])
            inter = jnp.maximum(xx2 - xx1, 0.0) * jnp.maximum(yy2 - yy1, 0.0)
            ba = (bx2 - bx1) * (by2 - by1)
            iou = inter / (ba + d["area"] - inter + 1e-9)
            sup = (iou > _IOU_THRESH) & (d["cpred"] == bcls)
            out.append(jnp.where(sup | pick, _NEG, s))
        return tuple(out)

    jax.lax.fori_loop(0, _MAX_PER_IMG, step, tuple(d["s0"] for d in imgs))


def kernel(x):
    b, n, c = x.shape
    xp = jnp.pad(x, ((0, 0), (0, _NP - n), (0, 0)))
    xt = xp.transpose(0, 2, 1).reshape(b, c, _R, _L)
    out = pl.pallas_call(
        _nms_body,
        grid=(b // _K,),
        in_specs=[pl.BlockSpec((_K, c, _R, _L), lambda i: (i, 0, 0, 0))],
        out_specs=pl.BlockSpec((_K, _MAX_PER_IMG, 8), lambda i: (i, 0, 0)),
        out_shape=jax.ShapeDtypeStruct((b, _MAX_PER_IMG, 8), jnp.float32),
        compiler_params=pltpu.CompilerParams(dimension_semantics=("parallel",)),
    )(xt)
    return out[:, :, :7]


# speedup vs baseline: 5.4804x; 1.0009x over previous
import jax
import jax.numpy as jnp
from jax.experimental import pallas as pl
from jax.experimental.pallas import tpu as pltpu

_CONF_THRESH = 0.25
_IOU_THRESH = 0.45
_MAX_PER_IMG = 100
_N = 5000
_NC = 80
_R = 40
_L = 128
_NP = _R * _L
_NEG = float("-inf")
_K = 4


_SX1, _SY1, _SX2, _SY2, _SAREA, _SOBJ, _SCCF, _SCPRED = range(8)


def _setup_one(x_ref, scr_ref, k):
    cx = x_ref[k, 0]
    cy = x_ref[k, 1]
    w = x_ref[k, 2]
    h = x_ref[k, 3]
    obj = x_ref[k, 4]

    x1 = cx - w / 2.0
    y1 = cy - h / 2.0
    x2 = cx + w / 2.0
    y2 = cy + h / 2.0
    area = (x2 - x1) * (y2 - y1)

    cls = x_ref[k, 5:5 + _NC]
    ccf = jnp.max(cls, axis=0)
    cidx = jax.lax.broadcasted_iota(jnp.int32, (_NC, _R, _L), 0)
    cpred = jnp.min(jnp.where(cls == ccf[None], cidx, 2**30), axis=0).astype(jnp.float32)

    score = obj * ccf
    ri = jax.lax.broadcasted_iota(jnp.int32, (_R, _L), 0)
    li = jax.lax.broadcasted_iota(jnp.int32, (_R, _L), 1)
    flat = ri * _L + li
    valid = flat < _N

    m0 = jnp.max(jnp.where(valid, score, _NEG), keepdims=True)
    conf = jnp.minimum(_CONF_THRESH, m0)
    s0 = jnp.where(valid & (score >= conf), score, _NEG)

    scr_ref[k, _SX1] = x1
    scr_ref[k, _SY1] = y1
    scr_ref[k, _SX2] = x2
    scr_ref[k, _SY2] = y2
    scr_ref[k, _SAREA] = area
    scr_ref[k, _SOBJ] = obj
    scr_ref[k, _SCCF] = ccf
    scr_ref[k, _SCPRED] = cpred
    return s0


def _nms_body(x_ref, o_ref, scr_ref):
    s0s = [_setup_one(x_ref, scr_ref, k) for k in range(_K)]
    lane8 = jax.lax.broadcasted_iota(jnp.int32, (1, 8), 1)
    ri = jax.lax.broadcasted_iota(jnp.int32, (_R, _L), 0)
    li = jax.lax.broadcasted_iota(jnp.int32, (_R, _L), 1)
    flat = ri * _L + li

    def step(t, ss):
        out = []
        for k in range(_K):
            s = ss[k]
            x1 = scr_ref[k, _SX1]
            y1 = scr_ref[k, _SY1]
            x2 = scr_ref[k, _SX2]
            y2 = scr_ref[k, _SY2]
            area = scr_ref[k, _SAREA]
            cpred = scr_ref[k, _SCPRED]
            m = jnp.max(s, keepdims=True)
            i = jnp.min(jnp.where(s == m, flat, 2**30), keepdims=True)
            pick = flat == i
            pf = pick.astype(jnp.float32)
            bx1 = jnp.sum(pf * x1, keepdims=True)
            by1 = jnp.sum(pf * y1, keepdims=True)
            bx2 = jnp.sum(pf * x2, keepdims=True)
            by2 = jnp.sum(pf * y2, keepdims=True)
            bobj = jnp.sum(pf * scr_ref[k, _SOBJ], keepdims=True)
            bccf = jnp.sum(pf * scr_ref[k, _SCCF], keepdims=True)
            bcls = jnp.sum(pf * cpred, keepdims=True)

            okf = jnp.where(m > _NEG, 1.0, 0.0)
            row = (jnp.where(lane8 == 0, bx1, 0.0)
                   + jnp.where(lane8 == 1, by1, 0.0)
                   + jnp.where(lane8 == 2, bx2, 0.0)
                   + jnp.where(lane8 == 3, by2, 0.0)
                   + jnp.where(lane8 == 4, bobj, 0.0)
                   + jnp.where(lane8 == 5, bccf, 0.0)
                   + jnp.where(lane8 == 6, bcls, 0.0)) * okf
            o_ref[k, pl.ds(t, 1), :] = row

            xx1 = jnp.maximum(bx1, x1)
            yy1 = jnp.maximum(by1, y1)
            xx2 = jnp.minimum(bx2, x2)
            yy2 = jnp.minimum(by2, y2)
            inter = jnp.maximum(xx2 - xx1, 0.0) * jnp.maximum(yy2 - yy1, 0.0)
            ba = (bx2 - bx1) * (by2 - by1)
            iou = inter / (ba + area - inter + 1e-9)
            sup = (iou > _IOU_THRESH) & (cpred == bcls)
            out.append(jnp.where(sup | pick, _NEG, s))
        return tuple(out)

    jax.lax.fori_loop(0, _MAX_PER_IMG, step, tuple(s0s))


def kernel(x):
    b, n, c = x.shape
    xp = jnp.pad(x, ((0, 0), (0, _NP - n), (0, 0)))
    xt = xp.transpose(0, 2, 1).reshape(b, c, _R, _L)
    out = pl.pallas_call(
        _nms_body,
        grid=(b // _K,),
        in_specs=[pl.BlockSpec((_K, c, _R, _L), lambda i: (i, 0, 0, 0))],
        out_specs=pl.BlockSpec((_K, _MAX_PER_IMG, 8), lambda i: (i, 0, 0)),
        out_shape=jax.ShapeDtypeStruct((b, _MAX_PER_IMG, 8), jnp.float32),
        scratch_shapes=[pltpu.VMEM((_K, 8, _R, _L), jnp.float32)],
        compiler_params=pltpu.CompilerParams(dimension_semantics=("parallel",)),
    )(xt)
    return out[:, :, :7]
